# baseline (speedup 1.0000x reference)
import functools

import jax
import jax.numpy as jnp
import numpy as np
from jax.experimental import pallas as pl
from jax.experimental.pallas import tpu as pltpu

B, N, C = 64, 8732, 21
TOTAL = B * N
CHUNKS = TOTAL // 128
S = 64
GRID = (CHUNKS + S - 1) // S

_ALPHA = 0.25
_OBJ_THRESH = 0.4

_f = np.arange(C * 128)
_SEGC = (_f[:, None] // C == np.arange(128)[None, :]).astype(np.float32)
_ETGT = _SEGC.T.copy()
_g = np.arange(4 * 128)
_SEG4 = (_g[:, None] // 4 == np.arange(128)[None, :]).astype(np.float32)
_h = np.arange(2 * 128)
_D0 = (_h[:, None] == 2 * np.arange(128)[None, :]).astype(np.float32)
_D1 = (_h[:, None] == 2 * np.arange(128)[None, :] + 1).astype(np.float32)
_CMOD = (_f % C).astype(np.float32).reshape(1, C * 128)


def _dot(a, b):
    return jax.lax.dot_general(a.astype(jnp.bfloat16), b,
                               (((1,), (0,)), ((), ())),
                               preferred_element_type=jnp.float32)


def _body(conf_ref, tgt_ref, osp_ref, ost_ref, loc1_ref, loc2_ref, loct_ref,
          cmod_ref, etgt_ref, segc_ref, seg4_ref, d0_ref, d1_ref, out_ref):
    pid = pl.program_id(0)

    @pl.when(pid == 0)
    def _init():
        for i in range(8):
            out_ref[i] = 0.0

    srow = jax.lax.broadcasted_iota(jnp.int32, (S, 128), 0) + pid * S
    valid = srow < CHUNKS
    maskf = valid.astype(jnp.float32)

    conf = conf_ref[...]
    tgt = tgt_ref[...]
    tgt_flat = _dot(tgt.astype(jnp.float32), etgt_ref[...])
    cmod = cmod_ref[...]
    conf_at_tgt = jnp.where(jnp.abs(cmod - tgt_flat) < 0.5, conf, 0.0)
    sumexp = _dot(jnp.exp(conf), segc_ref[...])
    xt = _dot(conf_at_tgt, segc_ref[...])
    ce = jnp.log(sumexp) - xt
    pos = tgt > 0

    osp = osp_ref[...]
    x0 = _dot(osp, d0_ref[...])
    x1 = _dot(osp, d1_ref[...])
    m = jnp.maximum(x0, x1)
    e0 = jnp.exp(x0 - m)
    e1 = jnp.exp(x1 - m)
    se = e0 + e1
    lse2 = m + jnp.log(se)
    p1 = e1 / se
    ost = ost_ref[...]
    xy = jnp.where(ost == 1, x1, x0)
    logpt = xy - lse2
    pt = jnp.exp(logpt)
    alpha_t = jnp.where(ost == 0, 1.0 - _ALPHA, _ALPHA)
    focal = -alpha_t * logpt * (1.0 - pt) * (1.0 - pt)
    focal_sum = jnp.sum(jnp.where(valid, focal, 0.0))
    pos_num = jnp.sum(jnp.where(jnp.logical_and(valid, ost > 0), 1.0, 0.0))

    os_pos = p1 > _OBJ_THRESH
    sel = jnp.where(jnp.logical_and(valid, jnp.logical_or(pos, os_pos)),
                    1.0, 0.0)
    ce_sum = jnp.sum(jnp.where(valid, ce, 0.0) * sel)
    sel_sum = jnp.sum(sel)

    loct = loct_ref[...]
    dd1 = loc1_ref[...] - loct
    dd2 = loc2_ref[...] - loct
    ad1 = jnp.abs(dd1)
    ad2 = jnp.abs(dd2)
    sl1 = jnp.where(ad1 < 1.0, 0.5 * dd1 * dd1, ad1 - 0.5)
    sl2 = jnp.where(ad2 < 1.0, 0.5 * dd2 * dd2, ad2 - 0.5)
    rows1 = _dot(sl1, seg4_ref[...])
    rows2 = _dot(sl2, seg4_ref[...])
    posf = jnp.where(pos, maskf, 0.0)
    l1_sum = jnp.sum(jnp.where(valid, rows1, 0.0) * posf)
    l2_sum = jnp.sum(jnp.where(valid, rows2, 0.0) * posf)
    reg_num = jnp.sum(posf)

    out_ref[0] += l1_sum
    out_ref[1] += l2_sum
    out_ref[2] += reg_num
    out_ref[3] += focal_sum
    out_ref[4] += pos_num
    out_ref[5] += ce_sum
    out_ref[6] += sel_sum

    @pl.when(pid == GRID - 1)
    def _finalize():
        l1 = out_ref[0]
        l2 = out_ref[1]
        rn = out_ref[2]
        fs = out_ref[3]
        pn = out_ref[4]
        cs = out_ref[5]
        ss = out_ref[6]
        loc_loss = (l2 * 0.5 + l1 * 0.35) / jnp.maximum(rn, 1.0)
        os_loss = fs * 10.0
        os_loss = jnp.where(pn > 0, os_loss / jnp.maximum(pn, 1.0),
                            os_loss / 500.0)
        conf_loss = cs / jnp.maximum(ss, 1.0)
        out_ref[0] = loc_loss
        out_ref[1] = os_loss
        out_ref[2] = conf_loss


@functools.partial(jax.jit, static_argnames=("interpret",))
def _fused(loc1, loc2, loct, conf, tgt, osp, ost, interpret=False):
    conf2 = conf.reshape(CHUNKS, C * 128)
    loc1r = loc1.reshape(CHUNKS, 512)
    loc2r = loc2.reshape(CHUNKS, 512)
    loctr = loct.reshape(CHUNKS, 512)
    tgt2 = tgt.reshape(CHUNKS, 128)
    osp2 = osp.reshape(CHUNKS, 256)
    ost2 = ost.reshape(CHUNKS, 128)

    def rowblock(width):
        return pl.BlockSpec((S, width), lambda i: (i, 0))

    def const(shape):
        return pl.BlockSpec(shape, lambda i: (0, 0))

    out = pl.pallas_call(
        _body,
        grid=(GRID,),
        in_specs=[
            rowblock(C * 128),
            rowblock(128),
            rowblock(256),
            rowblock(128),
            rowblock(512),
            rowblock(512),
            rowblock(512),
            const((1, C * 128)),
            const((128, C * 128)),
            const((C * 128, 128)),
            const((512, 128)),
            const((256, 128)),
            const((256, 128)),
        ],
        out_specs=pl.BlockSpec(memory_space=pltpu.SMEM),
        out_shape=jax.ShapeDtypeStruct((8,), jnp.float32),
        compiler_params=pltpu.CompilerParams(
            dimension_semantics=("arbitrary",),
        ),
        interpret=interpret,
    )(conf2, tgt2, osp2, ost2, loc1r, loc2r, loctr,
      jnp.asarray(_CMOD),
      jnp.asarray(_ETGT, dtype=jnp.bfloat16),
      jnp.asarray(_SEGC, dtype=jnp.bfloat16),
      jnp.asarray(_SEG4, dtype=jnp.bfloat16),
      jnp.asarray(_D0, dtype=jnp.bfloat16),
      jnp.asarray(_D1, dtype=jnp.bfloat16))
    return out[0], out[1], out[2]


def kernel(loc1_preds, loc2_preds, loc_targets, conf_preds, conf_targets,
           os_pred, os_target):
    return _fused(loc1_preds, loc2_preds, loc_targets, conf_preds,
                  conf_targets, os_pred, os_target)


# speedup vs baseline: 59.5920x; 59.5920x over previous
import functools

import jax
import jax.numpy as jnp
from jax.experimental import pallas as pl
from jax.experimental.pallas import tpu as pltpu

B, N, C = 64, 8732, 21
NB = 512
GRID = (N + NB - 1) // NB

_ALPHA = 0.25
_OBJ_THRESH = 0.4


def _body(conf_ref, tgt_ref, osp_ref, ost_ref, loc1_ref, loc2_ref, loct_ref,
          out_ref):
    pid = pl.program_id(0)

    @pl.when(pid == 0)
    def _init():
        for i in range(8):
            out_ref[i] = 0.0

    ncol = jax.lax.broadcasted_iota(jnp.int32, (B, NB), 1) + pid * NB
    valid = ncol < N
    maskf = valid.astype(jnp.float32)

    tgt = tgt_ref[...]
    pos = tgt > 0

    sumexp = jnp.zeros((B, NB), jnp.float32)
    xt = jnp.zeros((B, NB), jnp.float32)
    for c in range(C):
        plane = conf_ref[c]
        sumexp = sumexp + jnp.exp(plane)
        xt = jnp.where(tgt == c, plane, xt)
    ce = jnp.log(sumexp) - xt
    ce = jnp.where(valid, ce, 0.0)

    x0 = osp_ref[:, 0, :]
    x1 = osp_ref[:, 1, :]
    m = jnp.maximum(x0, x1)
    e0 = jnp.exp(x0 - m)
    e1 = jnp.exp(x1 - m)
    se = e0 + e1
    lse2 = m + jnp.log(se)
    p1 = e1 / se
    ost = ost_ref[...]
    xy = jnp.where(ost == 1, x1, x0)
    logpt = xy - lse2
    pt = jnp.exp(logpt)
    alpha_t = jnp.where(ost == 0, 1.0 - _ALPHA, _ALPHA)
    focal = -alpha_t * logpt * (1.0 - pt) * (1.0 - pt)
    focal_sum = jnp.sum(jnp.where(valid, focal, 0.0))
    pos_num = jnp.sum(jnp.where(jnp.logical_and(valid, ost > 0), 1.0, 0.0))

    os_pos = p1 > _OBJ_THRESH
    sel = jnp.where(jnp.logical_and(valid, jnp.logical_or(pos, os_pos)),
                    1.0, 0.0)
    ce_sum = jnp.sum(ce * sel)
    sel_sum = jnp.sum(sel)

    sl1_tot = jnp.zeros((B, NB), jnp.float32)
    sl2_tot = jnp.zeros((B, NB), jnp.float32)
    for q in range(4):
        t = loct_ref[:, q, :]
        d1 = loc1_ref[:, q, :] - t
        d2 = loc2_ref[:, q, :] - t
        a1 = jnp.abs(d1)
        a2 = jnp.abs(d2)
        sl1_tot = sl1_tot + jnp.where(a1 < 1.0, 0.5 * d1 * d1, a1 - 0.5)
        sl2_tot = sl2_tot + jnp.where(a2 < 1.0, 0.5 * d2 * d2, a2 - 0.5)
    posf = jnp.where(pos, maskf, 0.0)
    l1_sum = jnp.sum(jnp.where(valid, sl1_tot, 0.0) * posf)
    l2_sum = jnp.sum(jnp.where(valid, sl2_tot, 0.0) * posf)
    reg_num = jnp.sum(posf)

    out_ref[0] += l1_sum
    out_ref[1] += l2_sum
    out_ref[2] += reg_num
    out_ref[3] += focal_sum
    out_ref[4] += pos_num
    out_ref[5] += ce_sum
    out_ref[6] += sel_sum

    @pl.when(pid == GRID - 1)
    def _finalize():
        l1 = out_ref[0]
        l2 = out_ref[1]
        rn = out_ref[2]
        fs = out_ref[3]
        pn = out_ref[4]
        cs = out_ref[5]
        ss = out_ref[6]
        loc_loss = (l2 * 0.5 + l1 * 0.35) / jnp.maximum(rn, 1.0)
        os_loss = fs * 10.0
        os_loss = jnp.where(pn > 0, os_loss / jnp.maximum(pn, 1.0),
                            os_loss / 500.0)
        conf_loss = cs / jnp.maximum(ss, 1.0)
        out_ref[0] = loc_loss
        out_ref[1] = os_loss
        out_ref[2] = conf_loss


@functools.partial(jax.jit, static_argnames=("interpret",))
def _fused(loc1, loc2, loct, conf, tgt, osp, ost, interpret=False):
    confT = conf.transpose(2, 0, 1)
    loc1T = loc1.transpose(0, 2, 1)
    loc2T = loc2.transpose(0, 2, 1)
    loctT = loct.transpose(0, 2, 1)
    ospT = osp.transpose(0, 2, 1)

    out = pl.pallas_call(
        _body,
        grid=(GRID,),
        in_specs=[
            pl.BlockSpec((C, B, NB), lambda i: (0, 0, i)),
            pl.BlockSpec((B, NB), lambda i: (0, i)),
            pl.BlockSpec((B, 2, NB), lambda i: (0, 0, i)),
            pl.BlockSpec((B, NB), lambda i: (0, i)),
            pl.BlockSpec((B, 4, NB), lambda i: (0, 0, i)),
            pl.BlockSpec((B, 4, NB), lambda i: (0, 0, i)),
            pl.BlockSpec((B, 4, NB), lambda i: (0, 0, i)),
        ],
        out_specs=pl.BlockSpec(memory_space=pltpu.SMEM),
        out_shape=jax.ShapeDtypeStruct((8,), jnp.float32),
        compiler_params=pltpu.CompilerParams(
            dimension_semantics=("arbitrary",),
        ),
        interpret=interpret,
    )(confT, tgt, ospT, ost, loc1T, loc2T, loctT)
    return out[0], out[1], out[2]


def kernel(loc1_preds, loc2_preds, loc_targets, conf_preds, conf_targets,
           os_pred, os_target):
    return _fused(loc1_preds, loc2_preds, loc_targets, conf_preds,
                  conf_targets, os_pred, os_target)


# speedup vs baseline: 72.4815x; 1.2163x over previous
import functools

import jax
import jax.numpy as jnp
from jax.experimental import pallas as pl
from jax.experimental.pallas import tpu as pltpu

B, N, C = 64, 8732, 21
NB = 1024
GRID = (N + NB - 1) // NB

_ALPHA = 0.25
_OBJ_THRESH = 0.4


def _body(conf_ref, tgt_ref, osp_ref, ost_ref, loc1_ref, loc2_ref, loct_ref,
          out_ref):
    pid = pl.program_id(0)

    @pl.when(pid == 0)
    def _init():
        for i in range(8):
            out_ref[i] = 0.0

    ncol = jax.lax.broadcasted_iota(jnp.int32, (B, NB), 1) + pid * NB
    valid = ncol < N
    maskf = valid.astype(jnp.float32)

    tgt = tgt_ref[...]
    pos = tgt > 0

    sumexp = jnp.zeros((B, NB), jnp.float32)
    xt = jnp.zeros((B, NB), jnp.float32)
    for c in range(C):
        plane = conf_ref[c]
        sumexp = sumexp + jnp.exp(plane)
        xt = jnp.where(tgt == c, plane, xt)
    ce = jnp.log(sumexp) - xt
    ce = jnp.where(valid, ce, 0.0)

    x0 = osp_ref[:, 0, :]
    x1 = osp_ref[:, 1, :]
    m = jnp.maximum(x0, x1)
    e0 = jnp.exp(x0 - m)
    e1 = jnp.exp(x1 - m)
    se = e0 + e1
    lse2 = m + jnp.log(se)
    p1 = e1 / se
    ost = ost_ref[...]
    xy = jnp.where(ost == 1, x1, x0)
    logpt = xy - lse2
    pt = jnp.exp(logpt)
    alpha_t = jnp.where(ost == 0, 1.0 - _ALPHA, _ALPHA)
    focal = -alpha_t * logpt * (1.0 - pt) * (1.0 - pt)
    focal_sum = jnp.sum(jnp.where(valid, focal, 0.0))
    pos_num = jnp.sum(jnp.where(jnp.logical_and(valid, ost > 0), 1.0, 0.0))

    os_pos = p1 > _OBJ_THRESH
    sel = jnp.where(jnp.logical_and(valid, jnp.logical_or(pos, os_pos)),
                    1.0, 0.0)
    ce_sum = jnp.sum(ce * sel)
    sel_sum = jnp.sum(sel)

    sl1_tot = jnp.zeros((B, NB), jnp.float32)
    sl2_tot = jnp.zeros((B, NB), jnp.float32)
    for q in range(4):
        t = loct_ref[:, q, :]
        d1 = loc1_ref[:, q, :] - t
        d2 = loc2_ref[:, q, :] - t
        a1 = jnp.abs(d1)
        a2 = jnp.abs(d2)
        sl1_tot = sl1_tot + jnp.where(a1 < 1.0, 0.5 * d1 * d1, a1 - 0.5)
        sl2_tot = sl2_tot + jnp.where(a2 < 1.0, 0.5 * d2 * d2, a2 - 0.5)
    posf = jnp.where(pos, maskf, 0.0)
    l1_sum = jnp.sum(jnp.where(valid, sl1_tot, 0.0) * posf)
    l2_sum = jnp.sum(jnp.where(valid, sl2_tot, 0.0) * posf)
    reg_num = jnp.sum(posf)

    out_ref[0] += l1_sum
    out_ref[1] += l2_sum
    out_ref[2] += reg_num
    out_ref[3] += focal_sum
    out_ref[4] += pos_num
    out_ref[5] += ce_sum
    out_ref[6] += sel_sum

    @pl.when(pid == GRID - 1)
    def _finalize():
        l1 = out_ref[0]
        l2 = out_ref[1]
        rn = out_ref[2]
        fs = out_ref[3]
        pn = out_ref[4]
        cs = out_ref[5]
        ss = out_ref[6]
        loc_loss = (l2 * 0.5 + l1 * 0.35) / jnp.maximum(rn, 1.0)
        os_loss = fs * 10.0
        os_loss = jnp.where(pn > 0, os_loss / jnp.maximum(pn, 1.0),
                            os_loss / 500.0)
        conf_loss = cs / jnp.maximum(ss, 1.0)
        out_ref[0] = loc_loss
        out_ref[1] = os_loss
        out_ref[2] = conf_loss


@functools.partial(jax.jit, static_argnames=("interpret",))
def _fused(loc1, loc2, loct, conf, tgt, osp, ost, interpret=False):
    confT = conf.transpose(2, 0, 1)
    loc1T = loc1.transpose(0, 2, 1)
    loc2T = loc2.transpose(0, 2, 1)
    loctT = loct.transpose(0, 2, 1)
    ospT = osp.transpose(0, 2, 1)

    out = pl.pallas_call(
        _body,
        grid=(GRID,),
        in_specs=[
            pl.BlockSpec((C, B, NB), lambda i: (0, 0, i)),
            pl.BlockSpec((B, NB), lambda i: (0, i)),
            pl.BlockSpec((B, 2, NB), lambda i: (0, 0, i)),
            pl.BlockSpec((B, NB), lambda i: (0, i)),
            pl.BlockSpec((B, 4, NB), lambda i: (0, 0, i)),
            pl.BlockSpec((B, 4, NB), lambda i: (0, 0, i)),
            pl.BlockSpec((B, 4, NB), lambda i: (0, 0, i)),
        ],
        out_specs=pl.BlockSpec(memory_space=pltpu.SMEM),
        out_shape=jax.ShapeDtypeStruct((8,), jnp.float32),
        compiler_params=pltpu.CompilerParams(
            dimension_semantics=("arbitrary",),
        ),
        interpret=interpret,
    )(confT, tgt, ospT, ost, loc1T, loc2T, loctT)
    return out[0], out[1], out[2]


def kernel(loc1_preds, loc2_preds, loc_targets, conf_preds, conf_targets,
           os_pred, os_target):
    return _fused(loc1_preds, loc2_preds, loc_targets, conf_preds,
                  conf_targets, os_pred, os_target)
